# Initial kernel scaffold; baseline (speedup 1.0000x reference)
import jax
import jax.numpy as jnp
from jax.experimental import pallas as pl


def kernel(x, edge_index, W1, b1, W2, b2, g1W, g1b, g2W, g2b, cW, cb):
    raise NotImplementedError("write your pallas kernel here")


# speedup vs baseline: 8.1153x; 8.1153x over previous
import functools

import jax
import jax.numpy as jnp
from jax import lax
from jax.experimental import pallas as pl
from jax.experimental.pallas import tpu as pltpu
from jax.experimental.pallas import tpu_sc as plsc

NN = 10000
EE = 320000
DD = 128
HH = 128
EMB = 64
CC = 2
AL, BE, GA = 0.6, 0.4, 0.1

NPAD = 10240
DUMP = NPAD - 1
NC = 2
NS = 16
NW = NC * NS
EPT = NPAD
CHUNK = 128
NCHUNK = EPT // CHUNK
EPAD = NW * EPT
RPT = NPAD // NS


def _leaky(z):
    return jnp.where(z >= 0, z, 0.01 * z)


def _mesh():
    return plsc.VectorSubcoreMesh(core_axis_name="core", subcore_axis_name="subcore")


_SC_PARAMS = pltpu.CompilerParams(use_tc_tiling_on_sc=False)
_SC_PARAMS_NL = pltpu.CompilerParams(
    use_tc_tiling_on_sc=False, needs_layout_passes=False)


def _sc_degree(src3, dst3, zeros_n, ones_c):
    @functools.partial(
        pl.kernel,
        out_type=(
            jax.ShapeDtypeStruct((NC, NPAD), jnp.float32),
            jax.ShapeDtypeStruct((NC, NPAD), jnp.float32),
        ),
        mesh=_mesh(),
        compiler_params=_SC_PARAMS,
        scratch_types=[
            pltpu.VMEM((NCHUNK, CHUNK), jnp.int32),
            pltpu.VMEM((NCHUNK, CHUNK), jnp.int32),
            pltpu.VMEM((CHUNK,), jnp.float32),
            pltpu.VMEM_SHARED((NPAD,), jnp.float32),
            pltpu.VMEM_SHARED((NPAD,), jnp.float32),
        ],
    )
    def k(src_hbm, dst_hbm, z_hbm, ones_hbm, osrc_hbm, odst_hbm,
          src_v, dst_v, ones_v, csrc_sh, cdst_sh):
        cid = lax.axis_index("core")
        sid = lax.axis_index("subcore")
        wid = cid * NS + sid
        pltpu.sync_copy(src_hbm.at[wid], src_v)
        pltpu.sync_copy(dst_hbm.at[wid], dst_v)
        pltpu.sync_copy(ones_hbm, ones_v)
        r = pl.ds(sid * RPT, RPT)
        pltpu.sync_copy(z_hbm.at[r], csrc_sh.at[r])
        pltpu.sync_copy(z_hbm.at[r], cdst_sh.at[r])
        plsc.subcore_barrier()

        @pl.loop(0, NCHUNK)
        def _(j):
            pltpu.sync_copy(ones_v, csrc_sh.at[src_v.at[j]], add=True)
            pltpu.sync_copy(ones_v, cdst_sh.at[dst_v.at[j]], add=True)

        plsc.subcore_barrier()
        pltpu.sync_copy(csrc_sh.at[r], osrc_hbm.at[cid, r])
        pltpu.sync_copy(cdst_sh.at[r], odst_hbm.at[cid, r])

    return k(src3, dst3, zeros_n, ones_c)


def _sc_conv(src3, dst3, table, zeros_nd):
    @functools.partial(
        pl.kernel,
        out_type=jax.ShapeDtypeStruct((NC, NPAD, EMB), jnp.float32),
        mesh=_mesh(),
        compiler_params=_SC_PARAMS,
        scratch_types=[
            pltpu.VMEM((NCHUNK, CHUNK), jnp.int32),
            pltpu.VMEM((NCHUNK, CHUNK), jnp.int32),
            pltpu.VMEM((CHUNK, EMB), jnp.float32),
            pltpu.VMEM((CHUNK, EMB), jnp.float32),
            pltpu.VMEM_SHARED((NPAD, EMB), jnp.float32),
            pltpu.SemaphoreType.DMA,
            pltpu.SemaphoreType.DMA,
        ],
    )
    def k(src_hbm, dst_hbm, tab_hbm, z_hbm, out_hbm,
          src_v, dst_v, rows_a, rows_b, acc_sh, sem_a, sem_b):
        cid = lax.axis_index("core")
        sid = lax.axis_index("subcore")
        wid = cid * NS + sid
        pltpu.sync_copy(src_hbm.at[wid], src_v)
        pltpu.sync_copy(dst_hbm.at[wid], dst_v)
        r = pl.ds(sid * RPT, RPT)
        pltpu.sync_copy(z_hbm.at[r], acc_sh.at[r])
        plsc.subcore_barrier()

        pltpu.async_copy(tab_hbm.at[src_v.at[0]], rows_a, sem_a)

        @pl.loop(0, NCHUNK, step=2)
        def _(j):
            pltpu.async_copy(tab_hbm.at[src_v.at[j + 1]], rows_b, sem_b)
            pltpu.make_async_copy(tab_hbm.at[src_v.at[0]], rows_a, sem_a).wait()
            pltpu.sync_copy(rows_a, acc_sh.at[dst_v.at[j]], add=True)

            @pl.when(j + 2 < NCHUNK)
            def _():
                pltpu.async_copy(tab_hbm.at[src_v.at[j + 2]], rows_a, sem_a)

            pltpu.make_async_copy(tab_hbm.at[src_v.at[0]], rows_b, sem_b).wait()
            pltpu.sync_copy(rows_b, acc_sh.at[dst_v.at[j + 1]], add=True)

        plsc.subcore_barrier()
        pltpu.sync_copy(acc_sh.at[r], out_hbm.at[cid, r])

    return k(src3, dst3, table, zeros_nd)


def _sc_sim(src3, dst3, hn, zeros_n):
    @functools.partial(
        pl.kernel,
        out_type=jax.ShapeDtypeStruct((NC, NPAD), jnp.float32),
        mesh=_mesh(),
        compiler_params=_SC_PARAMS_NL,
        scratch_types=[
            pltpu.VMEM((NCHUNK, CHUNK), jnp.int32),
            pltpu.VMEM((NCHUNK, CHUNK), jnp.int32),
            pltpu.VMEM((CHUNK, EMB), jnp.float32),
            pltpu.VMEM((CHUNK, EMB), jnp.float32),
            pltpu.VMEM((CHUNK,), jnp.float32),
            pltpu.VMEM((16, 16), jnp.float32),
            pltpu.VMEM_SHARED((NPAD,), jnp.float32),
            pltpu.SemaphoreType.DMA,
            pltpu.SemaphoreType.DMA,
        ],
    )
    def k(src_hbm, dst_hbm, hn_hbm, z_hbm, out_hbm,
          src_v, dst_v, rows_s, rows_t, sims_v, tr_v, ss_sh, sem_a, sem_b):
        cid = lax.axis_index("core")
        sid = lax.axis_index("subcore")
        wid = cid * NS + sid
        pltpu.sync_copy(src_hbm.at[wid], src_v)
        pltpu.sync_copy(dst_hbm.at[wid], dst_v)
        r = pl.ds(sid * RPT, RPT)
        pltpu.sync_copy(z_hbm.at[r], ss_sh.at[r])
        plsc.subcore_barrier()
        lane = lax.iota(jnp.int32, 16)

        @pl.loop(0, NCHUNK)
        def _(j):
            pltpu.async_copy(hn_hbm.at[src_v.at[j]], rows_s, sem_a)
            pltpu.async_copy(hn_hbm.at[dst_v.at[j]], rows_t, sem_b)
            pltpu.make_async_copy(hn_hbm.at[src_v.at[j]], rows_s, sem_a).wait()
            pltpu.make_async_copy(hn_hbm.at[dst_v.at[j]], rows_t, sem_b).wait()

            @pl.loop(0, CHUNK // 16)
            def _(g):
                for e in range(16):
                    edge = g * 16 + e
                    ps = None
                    for c in range(EMB // 16):
                        a = rows_s[edge, pl.ds(c * 16, 16)]
                        b = rows_t[edge, pl.ds(c * 16, 16)]
                        ps = a * b if ps is None else ps + a * b
                    plsc.store_scatter(
                        tr_v, [lane, jnp.full((16,), e, jnp.int32)], ps)
                s = tr_v[0, :]
                for rr in range(1, 16):
                    s = s + tr_v[rr, :]
                sims_v[pl.ds(g * 16, 16)] = s

            pltpu.sync_copy(sims_v, ss_sh.at[src_v.at[j]], add=True)
            pltpu.sync_copy(sims_v, ss_sh.at[dst_v.at[j]], add=True)

        plsc.subcore_barrier()
        pltpu.sync_copy(ss_sh.at[r], out_hbm.at[cid, r])

    return k(src3, dst3, hn, zeros_n)


def _tc_mlp(xp, W1, b1r, W2, b2r, g1W):
    def body(x_ref, w1_ref, b1_ref, w2_ref, b2_ref, g1_ref, hw1_ref):
        h = _leaky(jnp.dot(x_ref[...], w1_ref[...],
                           preferred_element_type=jnp.float32) + b1_ref[...])
        h2 = _leaky(jnp.dot(h, w2_ref[...],
                            preferred_element_type=jnp.float32) + b2_ref[...])
        hw1_ref[...] = jnp.dot(h2, g1_ref[...],
                               preferred_element_type=jnp.float32)

    return pl.pallas_call(
        body,
        out_shape=jax.ShapeDtypeStruct((NPAD, EMB), jnp.float32),
    )(xp, W1, b1r, W2, b2r, g1W)


def _tc_prep(csrc0, csrc1, cdst0, cdst1, hw1):
    def body(cs0, cs1, cd0, cd1, hw1_ref, hws1_ref, dinv_ref, cnt_ref, gdd_ref):
        cdst = cd0[...] + cd1[...]
        cnt = cs0[...] + cs1[...] + cdst
        deg = cdst + 1.0
        dinv = lax.rsqrt(deg)
        hws1_ref[...] = dinv * hw1_ref[...]
        dinv_ref[...] = dinv
        cnt_ref[...] = cnt
        mask = (lax.broadcasted_iota(jnp.int32, (NPAD, 1), 0) < NN).astype(
            jnp.float32)
        cm = jnp.sum(cnt * mask) / NN
        cs = jnp.sqrt(jnp.sum((cnt - cm) ** 2 * mask) / (NN - 1))
        gdd_ref[...] = GA * (cnt - cm) / (cs + 1e-8)

    return pl.pallas_call(
        body,
        out_shape=(
            jax.ShapeDtypeStruct((NPAD, EMB), jnp.float32),
            jax.ShapeDtypeStruct((NPAD, 1), jnp.float32),
            jax.ShapeDtypeStruct((NPAD, 1), jnp.float32),
            jax.ShapeDtypeStruct((NPAD, 1), jnp.float32),
        ),
    )(csrc0, csrc1, cdst0, cdst1, hw1)


def _tc_mid(acc0, acc1, dinv, hw1, g1br, g2W):
    def body(a0, a1, dinv_ref, hw1_ref, g1b_ref, g2w_ref, hws2_ref, hw2_ref):
        dinv = dinv_ref[...]
        h3 = _leaky(dinv * (a0[...] + a1[...])
                    + dinv * dinv * hw1_ref[...] + g1b_ref[...])
        hw2 = jnp.dot(h3, g2w_ref[...], preferred_element_type=jnp.float32)
        hw2_ref[...] = hw2
        hws2_ref[...] = dinv * hw2

    return pl.pallas_call(
        body,
        out_shape=(
            jax.ShapeDtypeStruct((NPAD, EMB), jnp.float32),
            jax.ShapeDtypeStruct((NPAD, EMB), jnp.float32),
        ),
    )(acc0, acc1, dinv, hw1, g1br, g2W)


def _tc_final(acc0, acc1, dinv, hw2, g2br, cW, cbr, gdd):
    def body(a0, a1, dinv_ref, hw2_ref, g2b_ref, cw_ref, cb_ref, gdd_ref,
             hn_ref, base_ref):
        dinv = dinv_ref[...]
        h4 = _leaky(dinv * (a0[...] + a1[...])
                    + dinv * dinv * hw2_ref[...] + g2b_ref[...])
        logits = jnp.dot(h4, cw_ref[...],
                         preferred_element_type=jnp.float32) + cb_ref[...]
        m = jnp.max(logits, axis=1, keepdims=True)
        z = logits - m
        lse = jnp.log(jnp.sum(jnp.exp(z), axis=1, keepdims=True))
        logp = z - lse
        p = jnp.exp(logp)
        ent = -jnp.sum(p * logp, axis=1, keepdims=True)
        mask = (lax.broadcasted_iota(jnp.int32, (NPAD, 1), 0) < NN).astype(
            jnp.float32)
        em = jnp.sum(ent * mask) / NN
        es = jnp.sqrt(jnp.sum((ent - em) ** 2 * mask) / (NN - 1))
        std_ent = (ent - em) / (es + 1e-8)
        nrm = jnp.maximum(
            jnp.sqrt(jnp.sum(h4 * h4, axis=1, keepdims=True)), 1e-8)
        hn_ref[...] = h4 / nrm
        base_ref[...] = AL * std_ent + BE + gdd_ref[...]

    return pl.pallas_call(
        body,
        out_shape=(
            jax.ShapeDtypeStruct((NPAD, EMB), jnp.float32),
            jax.ShapeDtypeStruct((NPAD, 1), jnp.float32),
        ),
    )(acc0, acc1, dinv, hw2, g2br, cW, cbr, gdd)


def _tc_score(ssum0, ssum1, cnt, base):
    def body(s0, s1, cnt_ref, base_ref, out_ref):
        ssum = s0[...] + s1[...]
        cnt = cnt_ref[...]
        avg = jnp.where(cnt > 0, ssum / jnp.maximum(cnt, 1.0), 1.0)
        out_ref[...] = base_ref[...] - BE * avg

    return pl.pallas_call(
        body,
        out_shape=jax.ShapeDtypeStruct((NPAD, 1), jnp.float32),
    )(ssum0, ssum1, cnt, base)


def kernel(x, edge_index, W1, b1, W2, b2, g1W, g1b, g2W, g2b, cW, cb):
    xp = jnp.pad(x, ((0, NPAD - NN), (0, 0)))
    src = jnp.pad(edge_index[0], (0, EPAD - EE),
                  constant_values=DUMP).reshape(NW, NCHUNK, CHUNK)
    dst = jnp.pad(edge_index[1], (0, EPAD - EE),
                  constant_values=DUMP).reshape(NW, NCHUNK, CHUNK)
    zeros_n = jnp.zeros((NPAD,), jnp.float32)
    zeros_nd = jnp.zeros((NPAD, EMB), jnp.float32)
    ones_c = jnp.ones((CHUNK,), jnp.float32)
    b1r = b1.reshape(1, HH)
    b2r = b2.reshape(1, EMB)
    g1br = g1b.reshape(1, EMB)
    g2br = g2b.reshape(1, EMB)
    cbr = cb.reshape(1, CC)

    csrc_p, cdst_p = _sc_degree(src, dst, zeros_n, ones_c)
    hw1 = _tc_mlp(xp, W1, b1r, W2, b2r, g1W)
    hws1, dinv, cnt, gdd = _tc_prep(
        csrc_p[0].reshape(NPAD, 1), csrc_p[1].reshape(NPAD, 1),
        cdst_p[0].reshape(NPAD, 1), cdst_p[1].reshape(NPAD, 1), hw1)
    acc1 = _sc_conv(src, dst, hws1, zeros_nd)
    hws2, hw2 = _tc_mid(acc1[0], acc1[1], dinv, hw1, g1br, g2W)
    acc2 = _sc_conv(src, dst, hws2, zeros_nd)
    hn, base = _tc_final(acc2[0], acc2[1], dinv, hw2, g2br, cW, cbr, gdd)
    ss = _sc_sim(src, dst, hn, zeros_n)
    score = _tc_score(ss[0].reshape(NPAD, 1), ss[1].reshape(NPAD, 1),
                      cnt, base)
    return score[:NN, 0]


# speedup vs baseline: 8.8304x; 1.0881x over previous
import functools

import jax
import jax.numpy as jnp
from jax import lax
from jax.experimental import pallas as pl
from jax.experimental.pallas import tpu as pltpu
from jax.experimental.pallas import tpu_sc as plsc

NN = 10000
EE = 320000
DD = 128
HH = 128
EMB = 64
CC = 2
AL, BE, GA = 0.6, 0.4, 0.1

NPAD = 10240
DUMP = NPAD - 1
NC = 2
NS = 16
NW = NC * NS
EPT = NPAD
CHUNK = 128
NCHUNK = EPT // CHUNK
EPAD = NW * EPT
RPT = NPAD // NS


def _leaky(z):
    return jnp.where(z >= 0, z, 0.01 * z)


def _mesh():
    return plsc.VectorSubcoreMesh(core_axis_name="core", subcore_axis_name="subcore")


_SC_PARAMS = pltpu.CompilerParams(use_tc_tiling_on_sc=False)


def _sc_degree(src3, dst3, zeros_n, ones_c):
    @functools.partial(
        pl.kernel,
        out_type=(
            jax.ShapeDtypeStruct((NC, NPAD), jnp.float32),
            jax.ShapeDtypeStruct((NC, NPAD), jnp.float32),
        ),
        mesh=_mesh(),
        compiler_params=_SC_PARAMS,
        scratch_types=[
            pltpu.VMEM((NCHUNK, CHUNK), jnp.int32),
            pltpu.VMEM((NCHUNK, CHUNK), jnp.int32),
            pltpu.VMEM((CHUNK,), jnp.float32),
            pltpu.VMEM_SHARED((NPAD,), jnp.float32),
            pltpu.VMEM_SHARED((NPAD,), jnp.float32),
        ],
    )
    def k(src_hbm, dst_hbm, z_hbm, ones_hbm, osrc_hbm, odst_hbm,
          src_v, dst_v, ones_v, csrc_sh, cdst_sh):
        cid = lax.axis_index("core")
        sid = lax.axis_index("subcore")
        wid = cid * NS + sid
        pltpu.sync_copy(src_hbm.at[wid], src_v)
        pltpu.sync_copy(dst_hbm.at[wid], dst_v)
        pltpu.sync_copy(ones_hbm, ones_v)
        r = pl.ds(sid * RPT, RPT)
        pltpu.sync_copy(z_hbm.at[r], csrc_sh.at[r])
        pltpu.sync_copy(z_hbm.at[r], cdst_sh.at[r])
        plsc.subcore_barrier()

        @pl.loop(0, NCHUNK)
        def _(j):
            pltpu.sync_copy(ones_v, csrc_sh.at[src_v.at[j]], add=True)
            pltpu.sync_copy(ones_v, cdst_sh.at[dst_v.at[j]], add=True)

        plsc.subcore_barrier()
        pltpu.sync_copy(csrc_sh.at[r], osrc_hbm.at[cid, r])
        pltpu.sync_copy(cdst_sh.at[r], odst_hbm.at[cid, r])

    return k(src3, dst3, zeros_n, ones_c)


def _sc_conv(src3, dst3, table, zeros_nd):
    @functools.partial(
        pl.kernel,
        out_type=jax.ShapeDtypeStruct((NC, NPAD, EMB), jnp.float32),
        mesh=_mesh(),
        compiler_params=_SC_PARAMS,
        scratch_types=[
            pltpu.VMEM((NCHUNK, CHUNK), jnp.int32),
            pltpu.VMEM((NCHUNK, CHUNK), jnp.int32),
            pltpu.VMEM((CHUNK, EMB), jnp.float32),
            pltpu.VMEM((CHUNK, EMB), jnp.float32),
            pltpu.VMEM_SHARED((NPAD, EMB), jnp.float32),
            pltpu.SemaphoreType.DMA,
            pltpu.SemaphoreType.DMA,
        ],
    )
    def k(src_hbm, dst_hbm, tab_hbm, z_hbm, out_hbm,
          src_v, dst_v, rows_a, rows_b, acc_sh, sem_a, sem_b):
        cid = lax.axis_index("core")
        sid = lax.axis_index("subcore")
        wid = cid * NS + sid
        pltpu.sync_copy(src_hbm.at[wid], src_v)
        pltpu.sync_copy(dst_hbm.at[wid], dst_v)
        r = pl.ds(sid * RPT, RPT)
        pltpu.sync_copy(z_hbm.at[r], acc_sh.at[r])
        plsc.subcore_barrier()

        pltpu.async_copy(tab_hbm.at[src_v.at[0]], rows_a, sem_a)

        @pl.loop(0, NCHUNK, step=2)
        def _(j):
            pltpu.async_copy(tab_hbm.at[src_v.at[j + 1]], rows_b, sem_b)
            pltpu.make_async_copy(tab_hbm.at[src_v.at[0]], rows_a, sem_a).wait()
            pltpu.sync_copy(rows_a, acc_sh.at[dst_v.at[j]], add=True)

            @pl.when(j + 2 < NCHUNK)
            def _():
                pltpu.async_copy(tab_hbm.at[src_v.at[j + 2]], rows_a, sem_a)

            pltpu.make_async_copy(tab_hbm.at[src_v.at[0]], rows_b, sem_b).wait()
            pltpu.sync_copy(rows_b, acc_sh.at[dst_v.at[j + 1]], add=True)

        plsc.subcore_barrier()
        pltpu.sync_copy(acc_sh.at[r], out_hbm.at[cid, r])

    return k(src3, dst3, table, zeros_nd)


def _sc_sim(src3, dst3, hn, zeros_nd):
    @functools.partial(
        pl.kernel,
        out_type=jax.ShapeDtypeStruct((NC, NPAD, EMB), jnp.float32),
        mesh=_mesh(),
        compiler_params=_SC_PARAMS,
        scratch_types=[
            pltpu.VMEM((NCHUNK, CHUNK), jnp.int32),
            pltpu.VMEM((NCHUNK, CHUNK), jnp.int32),
            pltpu.VMEM((CHUNK, EMB), jnp.float32),
            pltpu.VMEM((CHUNK, EMB), jnp.float32),
            pltpu.VMEM((CHUNK, EMB), jnp.float32),
            pltpu.VMEM((CHUNK, EMB), jnp.float32),
            pltpu.VMEM_SHARED((NPAD, EMB), jnp.float32),
            pltpu.SemaphoreType.DMA,
            pltpu.SemaphoreType.DMA,
            pltpu.SemaphoreType.DMA,
            pltpu.SemaphoreType.DMA,
        ],
    )
    def k(src_hbm, dst_hbm, hn_hbm, z_hbm, out_hbm,
          src_v, dst_v, rows_sa, rows_sb, rows_ta, rows_tb, agg_sh,
          sem_sa, sem_sb, sem_ta, sem_tb):
        cid = lax.axis_index("core")
        sid = lax.axis_index("subcore")
        wid = cid * NS + sid
        pltpu.sync_copy(src_hbm.at[wid], src_v)
        pltpu.sync_copy(dst_hbm.at[wid], dst_v)
        r = pl.ds(sid * RPT, RPT)
        pltpu.sync_copy(z_hbm.at[r], agg_sh.at[r])
        plsc.subcore_barrier()

        pltpu.async_copy(hn_hbm.at[src_v.at[0]], rows_sa, sem_sa)
        pltpu.async_copy(hn_hbm.at[dst_v.at[0]], rows_ta, sem_ta)

        @pl.loop(0, NCHUNK, step=2)
        def _(j):
            pltpu.async_copy(hn_hbm.at[src_v.at[j + 1]], rows_sb, sem_sb)
            pltpu.async_copy(hn_hbm.at[dst_v.at[j + 1]], rows_tb, sem_tb)
            pltpu.make_async_copy(hn_hbm.at[src_v.at[0]], rows_sa, sem_sa).wait()
            pltpu.make_async_copy(hn_hbm.at[dst_v.at[0]], rows_ta, sem_ta).wait()
            pltpu.sync_copy(rows_sa, agg_sh.at[dst_v.at[j]], add=True)
            pltpu.sync_copy(rows_ta, agg_sh.at[src_v.at[j]], add=True)

            @pl.when(j + 2 < NCHUNK)
            def _():
                pltpu.async_copy(hn_hbm.at[src_v.at[j + 2]], rows_sa, sem_sa)
                pltpu.async_copy(hn_hbm.at[dst_v.at[j + 2]], rows_ta, sem_ta)

            pltpu.make_async_copy(hn_hbm.at[src_v.at[0]], rows_sb, sem_sb).wait()
            pltpu.make_async_copy(hn_hbm.at[dst_v.at[0]], rows_tb, sem_tb).wait()
            pltpu.sync_copy(rows_sb, agg_sh.at[dst_v.at[j + 1]], add=True)
            pltpu.sync_copy(rows_tb, agg_sh.at[src_v.at[j + 1]], add=True)

        plsc.subcore_barrier()
        pltpu.sync_copy(agg_sh.at[r], out_hbm.at[cid, r])

    return k(src3, dst3, hn, zeros_nd)


def _tc_mlp(xp, W1, b1r, W2, b2r, g1W):
    def body(x_ref, w1_ref, b1_ref, w2_ref, b2_ref, g1_ref, hw1_ref):
        h = _leaky(jnp.dot(x_ref[...], w1_ref[...],
                           preferred_element_type=jnp.float32) + b1_ref[...])
        h2 = _leaky(jnp.dot(h, w2_ref[...],
                            preferred_element_type=jnp.float32) + b2_ref[...])
        hw1_ref[...] = jnp.dot(h2, g1_ref[...],
                               preferred_element_type=jnp.float32)

    return pl.pallas_call(
        body,
        out_shape=jax.ShapeDtypeStruct((NPAD, EMB), jnp.float32),
    )(xp, W1, b1r, W2, b2r, g1W)


def _tc_prep(csrc0, csrc1, cdst0, cdst1, hw1):
    def body(cs0, cs1, cd0, cd1, hw1_ref, hws1_ref, dinv_ref, cnt_ref, gdd_ref):
        cdst = cd0[...] + cd1[...]
        cnt = cs0[...] + cs1[...] + cdst
        deg = cdst + 1.0
        dinv = lax.rsqrt(deg)
        hws1_ref[...] = dinv * hw1_ref[...]
        dinv_ref[...] = dinv
        cnt_ref[...] = cnt
        mask = (lax.broadcasted_iota(jnp.int32, (NPAD, 1), 0) < NN).astype(
            jnp.float32)
        cm = jnp.sum(cnt * mask) / NN
        cs = jnp.sqrt(jnp.sum((cnt - cm) ** 2 * mask) / (NN - 1))
        gdd_ref[...] = GA * (cnt - cm) / (cs + 1e-8)

    return pl.pallas_call(
        body,
        out_shape=(
            jax.ShapeDtypeStruct((NPAD, EMB), jnp.float32),
            jax.ShapeDtypeStruct((NPAD, 1), jnp.float32),
            jax.ShapeDtypeStruct((NPAD, 1), jnp.float32),
            jax.ShapeDtypeStruct((NPAD, 1), jnp.float32),
        ),
    )(csrc0, csrc1, cdst0, cdst1, hw1)


def _tc_mid(acc0, acc1, dinv, hw1, g1br, g2W):
    def body(a0, a1, dinv_ref, hw1_ref, g1b_ref, g2w_ref, hws2_ref, hw2_ref):
        dinv = dinv_ref[...]
        h3 = _leaky(dinv * (a0[...] + a1[...])
                    + dinv * dinv * hw1_ref[...] + g1b_ref[...])
        hw2 = jnp.dot(h3, g2w_ref[...], preferred_element_type=jnp.float32)
        hw2_ref[...] = hw2
        hws2_ref[...] = dinv * hw2

    return pl.pallas_call(
        body,
        out_shape=(
            jax.ShapeDtypeStruct((NPAD, EMB), jnp.float32),
            jax.ShapeDtypeStruct((NPAD, EMB), jnp.float32),
        ),
    )(acc0, acc1, dinv, hw1, g1br, g2W)


def _tc_final(acc0, acc1, dinv, hw2, g2br, cW, cbr, gdd):
    def body(a0, a1, dinv_ref, hw2_ref, g2b_ref, cw_ref, cb_ref, gdd_ref,
             hn_ref, base_ref):
        dinv = dinv_ref[...]
        h4 = _leaky(dinv * (a0[...] + a1[...])
                    + dinv * dinv * hw2_ref[...] + g2b_ref[...])
        logits = jnp.dot(h4, cw_ref[...],
                         preferred_element_type=jnp.float32) + cb_ref[...]
        m = jnp.max(logits, axis=1, keepdims=True)
        z = logits - m
        lse = jnp.log(jnp.sum(jnp.exp(z), axis=1, keepdims=True))
        logp = z - lse
        p = jnp.exp(logp)
        ent = -jnp.sum(p * logp, axis=1, keepdims=True)
        mask = (lax.broadcasted_iota(jnp.int32, (NPAD, 1), 0) < NN).astype(
            jnp.float32)
        em = jnp.sum(ent * mask) / NN
        es = jnp.sqrt(jnp.sum((ent - em) ** 2 * mask) / (NN - 1))
        std_ent = (ent - em) / (es + 1e-8)
        nrm = jnp.maximum(
            jnp.sqrt(jnp.sum(h4 * h4, axis=1, keepdims=True)), 1e-8)
        hn_ref[...] = h4 / nrm
        base_ref[...] = AL * std_ent + BE + gdd_ref[...]

    return pl.pallas_call(
        body,
        out_shape=(
            jax.ShapeDtypeStruct((NPAD, EMB), jnp.float32),
            jax.ShapeDtypeStruct((NPAD, 1), jnp.float32),
        ),
    )(acc0, acc1, dinv, hw2, g2br, cW, cbr, gdd)


def _tc_score(agg0, agg1, hn, cnt, base):
    def body(a0, a1, hn_ref, cnt_ref, base_ref, out_ref):
        ssum = jnp.sum(hn_ref[...] * (a0[...] + a1[...]), axis=1,
                       keepdims=True)
        cnt = cnt_ref[...]
        avg = jnp.where(cnt > 0, ssum / jnp.maximum(cnt, 1.0), 1.0)
        out_ref[...] = base_ref[...] - BE * avg

    return pl.pallas_call(
        body,
        out_shape=jax.ShapeDtypeStruct((NPAD, 1), jnp.float32),
    )(agg0, agg1, hn, cnt, base)


def kernel(x, edge_index, W1, b1, W2, b2, g1W, g1b, g2W, g2b, cW, cb):
    xp = jnp.pad(x, ((0, NPAD - NN), (0, 0)))
    src = jnp.pad(edge_index[0], (0, EPAD - EE),
                  constant_values=DUMP).reshape(NW, NCHUNK, CHUNK)
    dst = jnp.pad(edge_index[1], (0, EPAD - EE),
                  constant_values=DUMP).reshape(NW, NCHUNK, CHUNK)
    zeros_n = jnp.zeros((NPAD,), jnp.float32)
    zeros_nd = jnp.zeros((NPAD, EMB), jnp.float32)
    ones_c = jnp.ones((CHUNK,), jnp.float32)
    b1r = b1.reshape(1, HH)
    b2r = b2.reshape(1, EMB)
    g1br = g1b.reshape(1, EMB)
    g2br = g2b.reshape(1, EMB)
    cbr = cb.reshape(1, CC)

    csrc_p, cdst_p = _sc_degree(src, dst, zeros_n, ones_c)
    hw1 = _tc_mlp(xp, W1, b1r, W2, b2r, g1W)
    hws1, dinv, cnt, gdd = _tc_prep(
        csrc_p[0].reshape(NPAD, 1), csrc_p[1].reshape(NPAD, 1),
        cdst_p[0].reshape(NPAD, 1), cdst_p[1].reshape(NPAD, 1), hw1)
    acc1 = _sc_conv(src, dst, hws1, zeros_nd)
    hws2, hw2 = _tc_mid(acc1[0], acc1[1], dinv, hw1, g1br, g2W)
    acc2 = _sc_conv(src, dst, hws2, zeros_nd)
    hn, base = _tc_final(acc2[0], acc2[1], dinv, hw2, g2br, cW, cbr, gdd)
    agg = _sc_sim(src, dst, hn, zeros_nd)
    score = _tc_score(agg[0], agg[1], hn, cnt, base)
    return score[:NN, 0]


# speedup vs baseline: 25.3025x; 2.8654x over previous
import functools

import jax
import jax.numpy as jnp
from jax import lax
from jax.experimental import pallas as pl
from jax.experimental.pallas import tpu as pltpu
from jax.experimental.pallas import tpu_sc as plsc

NN = 10000
EE = 320000
DD = 128
HH = 128
EMB = 64
CC = 2
AL, BE, GA = 0.6, 0.4, 0.1

NPAD = 10240
DUMP = NPAD - 1
NC = 2
NS = 16
NW = NC * NS
EPT = NPAD
CHUNK = 128
NCHUNK = EPT // CHUNK
EPAD = NW * EPT
RPT = NPAD // NS


def _leaky(z):
    return jnp.where(z >= 0, z, 0.01 * z)


def _mesh():
    return plsc.VectorSubcoreMesh(core_axis_name="core", subcore_axis_name="subcore")


_SC_PARAMS = pltpu.CompilerParams(use_tc_tiling_on_sc=False)


def _sc_degree(src3, dst3, zeros_n, ones_c):
    @functools.partial(
        pl.kernel,
        out_type=(
            jax.ShapeDtypeStruct((NC, NPAD), jnp.float32),
            jax.ShapeDtypeStruct((NC, NPAD), jnp.float32),
        ),
        mesh=_mesh(),
        compiler_params=_SC_PARAMS,
        scratch_types=[
            pltpu.VMEM((NCHUNK, CHUNK), jnp.int32),
            pltpu.VMEM((NCHUNK, CHUNK), jnp.int32),
            pltpu.VMEM((CHUNK,), jnp.float32),
            pltpu.VMEM_SHARED((NPAD,), jnp.float32),
            pltpu.VMEM_SHARED((NPAD,), jnp.float32),
        ],
    )
    def k(src_hbm, dst_hbm, z_hbm, ones_hbm, osrc_hbm, odst_hbm,
          src_v, dst_v, ones_v, csrc_sh, cdst_sh):
        cid = lax.axis_index("core")
        sid = lax.axis_index("subcore")
        wid = cid * NS + sid
        pltpu.sync_copy(src_hbm.at[wid], src_v)
        pltpu.sync_copy(dst_hbm.at[wid], dst_v)
        pltpu.sync_copy(ones_hbm, ones_v)
        r = pl.ds(sid * RPT, RPT)
        pltpu.sync_copy(z_hbm.at[r], csrc_sh.at[r])
        pltpu.sync_copy(z_hbm.at[r], cdst_sh.at[r])
        plsc.subcore_barrier()

        @pl.loop(0, NCHUNK)
        def _(j):
            pltpu.sync_copy(ones_v, csrc_sh.at[src_v.at[j]], add=True)
            pltpu.sync_copy(ones_v, cdst_sh.at[dst_v.at[j]], add=True)

        plsc.subcore_barrier()
        pltpu.sync_copy(csrc_sh.at[r], osrc_hbm.at[cid, r])
        pltpu.sync_copy(cdst_sh.at[r], odst_hbm.at[cid, r])

    return k(src3, dst3, zeros_n, ones_c)


def _sc_conv(src3, dst3, table, zeros_nd):
    @functools.partial(
        pl.kernel,
        out_type=jax.ShapeDtypeStruct((NC, NPAD, EMB), jnp.float32),
        mesh=_mesh(),
        compiler_params=_SC_PARAMS,
        scratch_types=[
            pltpu.VMEM((NCHUNK, CHUNK), jnp.int32),
            pltpu.VMEM((NCHUNK, CHUNK), jnp.int32),
            pltpu.VMEM((CHUNK, EMB), jnp.float32),
            pltpu.VMEM((CHUNK, EMB), jnp.float32),
            pltpu.VMEM_SHARED((NPAD, EMB), jnp.float32),
            pltpu.SemaphoreType.DMA,
            pltpu.SemaphoreType.DMA,
        ],
    )
    def k(src_hbm, dst_hbm, tab_hbm, z_hbm, out_hbm,
          src_v, dst_v, rows_a, rows_b, acc_sh, sem_a, sem_b):
        cid = lax.axis_index("core")
        sid = lax.axis_index("subcore")
        wid = cid * NS + sid
        pltpu.sync_copy(src_hbm.at[wid], src_v)
        pltpu.sync_copy(dst_hbm.at[wid], dst_v)
        r = pl.ds(sid * RPT, RPT)
        pltpu.sync_copy(z_hbm.at[r], acc_sh.at[r])
        plsc.subcore_barrier()

        pltpu.async_copy(tab_hbm.at[src_v.at[0]], rows_a, sem_a)

        @pl.loop(0, NCHUNK, step=2)
        def _(j):
            pltpu.async_copy(tab_hbm.at[src_v.at[j + 1]], rows_b, sem_b)
            pltpu.make_async_copy(tab_hbm.at[src_v.at[0]], rows_a, sem_a).wait()
            pltpu.sync_copy(rows_a, acc_sh.at[dst_v.at[j]], add=True)

            @pl.when(j + 2 < NCHUNK)
            def _():
                pltpu.async_copy(tab_hbm.at[src_v.at[j + 2]], rows_a, sem_a)

            pltpu.make_async_copy(tab_hbm.at[src_v.at[0]], rows_b, sem_b).wait()
            pltpu.sync_copy(rows_b, acc_sh.at[dst_v.at[j + 1]], add=True)

        plsc.subcore_barrier()
        pltpu.sync_copy(acc_sh.at[r], out_hbm.at[cid, r])

    return k(src3, dst3, table, zeros_nd)


def _sc_sim(src3, dst3, hn, zeros_nd):
    @functools.partial(
        pl.kernel,
        out_type=jax.ShapeDtypeStruct((NC, NPAD, EMB), jnp.float32),
        mesh=_mesh(),
        compiler_params=_SC_PARAMS,
        scratch_types=[
            pltpu.VMEM((NCHUNK, CHUNK), jnp.int32),
            pltpu.VMEM((NCHUNK, CHUNK), jnp.int32),
            pltpu.VMEM((CHUNK, EMB), jnp.float32),
            pltpu.VMEM((CHUNK, EMB), jnp.float32),
            pltpu.VMEM((CHUNK, EMB), jnp.float32),
            pltpu.VMEM((CHUNK, EMB), jnp.float32),
            pltpu.VMEM_SHARED((NPAD, EMB), jnp.float32),
            pltpu.SemaphoreType.DMA,
            pltpu.SemaphoreType.DMA,
            pltpu.SemaphoreType.DMA,
            pltpu.SemaphoreType.DMA,
        ],
    )
    def k(src_hbm, dst_hbm, hn_hbm, z_hbm, out_hbm,
          src_v, dst_v, rows_sa, rows_sb, rows_ta, rows_tb, agg_sh,
          sem_sa, sem_sb, sem_ta, sem_tb):
        cid = lax.axis_index("core")
        sid = lax.axis_index("subcore")
        wid = cid * NS + sid
        pltpu.sync_copy(src_hbm.at[wid], src_v)
        pltpu.sync_copy(dst_hbm.at[wid], dst_v)
        r = pl.ds(sid * RPT, RPT)
        pltpu.sync_copy(z_hbm.at[r], agg_sh.at[r])
        plsc.subcore_barrier()

        pltpu.async_copy(hn_hbm.at[src_v.at[0]], rows_sa, sem_sa)
        pltpu.async_copy(hn_hbm.at[dst_v.at[0]], rows_ta, sem_ta)

        @pl.loop(0, NCHUNK, step=2)
        def _(j):
            pltpu.async_copy(hn_hbm.at[src_v.at[j + 1]], rows_sb, sem_sb)
            pltpu.async_copy(hn_hbm.at[dst_v.at[j + 1]], rows_tb, sem_tb)
            pltpu.make_async_copy(hn_hbm.at[src_v.at[0]], rows_sa, sem_sa).wait()
            pltpu.make_async_copy(hn_hbm.at[dst_v.at[0]], rows_ta, sem_ta).wait()
            pltpu.sync_copy(rows_sa, agg_sh.at[dst_v.at[j]], add=True)
            pltpu.sync_copy(rows_ta, agg_sh.at[src_v.at[j]], add=True)

            @pl.when(j + 2 < NCHUNK)
            def _():
                pltpu.async_copy(hn_hbm.at[src_v.at[j + 2]], rows_sa, sem_sa)
                pltpu.async_copy(hn_hbm.at[dst_v.at[j + 2]], rows_ta, sem_ta)

            pltpu.make_async_copy(hn_hbm.at[src_v.at[0]], rows_sb, sem_sb).wait()
            pltpu.make_async_copy(hn_hbm.at[dst_v.at[0]], rows_tb, sem_tb).wait()
            pltpu.sync_copy(rows_sb, agg_sh.at[dst_v.at[j + 1]], add=True)
            pltpu.sync_copy(rows_tb, agg_sh.at[src_v.at[j + 1]], add=True)

        plsc.subcore_barrier()
        pltpu.sync_copy(agg_sh.at[r], out_hbm.at[cid, r])

    return k(src3, dst3, hn, zeros_nd)


def _tc_mlp(xp, W1, b1r, W2, b2r, g1W):
    def body(x_ref, w1_ref, b1_ref, w2_ref, b2_ref, g1_ref, hw1_ref):
        h = _leaky(jnp.dot(x_ref[...], w1_ref[...],
                           preferred_element_type=jnp.float32) + b1_ref[...])
        h2 = _leaky(jnp.dot(h, w2_ref[...],
                            preferred_element_type=jnp.float32) + b2_ref[...])
        hw1_ref[...] = jnp.dot(h2, g1_ref[...],
                               preferred_element_type=jnp.float32)

    return pl.pallas_call(
        body,
        out_shape=jax.ShapeDtypeStruct((NPAD, EMB), jnp.float32),
    )(xp, W1, b1r, W2, b2r, g1W)


def _tc_prep(csrc0, csrc1, cdst0, cdst1, hw1):
    def body(cs0, cs1, cd0, cd1, hw1_ref, hws1_ref, dinv_ref, cnt_ref, gdd_ref):
        cdst = cd0[...] + cd1[...]
        cnt = cs0[...] + cs1[...] + cdst
        deg = cdst + 1.0
        dinv = lax.rsqrt(deg)
        hws1_ref[...] = dinv * hw1_ref[...]
        dinv_ref[...] = dinv
        cnt_ref[...] = cnt
        mask = (lax.broadcasted_iota(jnp.int32, (NPAD, 1), 0) < NN).astype(
            jnp.float32)
        cm = jnp.sum(cnt * mask) / NN
        cs = jnp.sqrt(jnp.sum((cnt - cm) ** 2 * mask) / (NN - 1))
        gdd_ref[...] = GA * (cnt - cm) / (cs + 1e-8)

    return pl.pallas_call(
        body,
        out_shape=(
            jax.ShapeDtypeStruct((NPAD, EMB), jnp.float32),
            jax.ShapeDtypeStruct((NPAD, 1), jnp.float32),
            jax.ShapeDtypeStruct((NPAD, 1), jnp.float32),
            jax.ShapeDtypeStruct((NPAD, 1), jnp.float32),
        ),
    )(csrc0, csrc1, cdst0, cdst1, hw1)


def _tc_mid(acc0, acc1, dinv, hw1, g1br, g2W):
    def body(a0, a1, dinv_ref, hw1_ref, g1b_ref, g2w_ref, hws2_ref, hw2_ref):
        dinv = dinv_ref[...]
        h3 = _leaky(dinv * (a0[...] + a1[...])
                    + dinv * dinv * hw1_ref[...] + g1b_ref[...])
        hw2 = jnp.dot(h3, g2w_ref[...], preferred_element_type=jnp.float32)
        hw2_ref[...] = hw2
        hws2_ref[...] = dinv * hw2

    return pl.pallas_call(
        body,
        out_shape=(
            jax.ShapeDtypeStruct((NPAD, EMB), jnp.float32),
            jax.ShapeDtypeStruct((NPAD, EMB), jnp.float32),
        ),
    )(acc0, acc1, dinv, hw1, g1br, g2W)


def _tc_final(acc0, acc1, dinv, hw2, g2br, cW, cbr, gdd):
    def body(a0, a1, dinv_ref, hw2_ref, g2b_ref, cw_ref, cb_ref, gdd_ref,
             hn_ref, base_ref):
        dinv = dinv_ref[...]
        h4 = _leaky(dinv * (a0[...] + a1[...])
                    + dinv * dinv * hw2_ref[...] + g2b_ref[...])
        logits = jnp.dot(h4, cw_ref[...],
                         preferred_element_type=jnp.float32) + cb_ref[...]
        m = jnp.max(logits, axis=1, keepdims=True)
        z = logits - m
        lse = jnp.log(jnp.sum(jnp.exp(z), axis=1, keepdims=True))
        logp = z - lse
        p = jnp.exp(logp)
        ent = -jnp.sum(p * logp, axis=1, keepdims=True)
        mask = (lax.broadcasted_iota(jnp.int32, (NPAD, 1), 0) < NN).astype(
            jnp.float32)
        em = jnp.sum(ent * mask) / NN
        es = jnp.sqrt(jnp.sum((ent - em) ** 2 * mask) / (NN - 1))
        std_ent = (ent - em) / (es + 1e-8)
        nrm = jnp.maximum(
            jnp.sqrt(jnp.sum(h4 * h4, axis=1, keepdims=True)), 1e-8)
        hn_ref[...] = h4 / nrm
        base_ref[...] = AL * std_ent + BE + gdd_ref[...]

    return pl.pallas_call(
        body,
        out_shape=(
            jax.ShapeDtypeStruct((NPAD, EMB), jnp.float32),
            jax.ShapeDtypeStruct((NPAD, 1), jnp.float32),
        ),
    )(acc0, acc1, dinv, hw2, g2br, cW, cbr, gdd)


def _tc_score(agg0, agg1, hn, cnt, base):
    def body(a0, a1, hn_ref, cnt_ref, base_ref, out_ref):
        ssum = jnp.sum(hn_ref[...] * (a0[...] + a1[...]), axis=1,
                       keepdims=True)
        cnt = cnt_ref[...]
        avg = jnp.where(cnt > 0, ssum / jnp.maximum(cnt, 1.0), 1.0)
        out_ref[...] = base_ref[...] - BE * avg

    return pl.pallas_call(
        body,
        out_shape=jax.ShapeDtypeStruct((NPAD, 1), jnp.float32),
    )(agg0, agg1, hn, cnt, base)


def kernel(x, edge_index, W1, b1, W2, b2, g1W, g1b, g2W, g2b, cW, cb):
    xp = jnp.pad(x, ((0, NPAD - NN), (0, 0)))
    pad_idx = NN + jnp.arange(EPAD - EE, dtype=jnp.int32) % (NPAD - NN)
    src = jnp.concatenate([edge_index[0], pad_idx]).reshape(NW, NCHUNK, CHUNK)
    dst = jnp.concatenate([edge_index[1], pad_idx]).reshape(NW, NCHUNK, CHUNK)
    zeros_n = jnp.zeros((NPAD,), jnp.float32)
    zeros_nd = jnp.zeros((NPAD, EMB), jnp.float32)
    ones_c = jnp.ones((CHUNK,), jnp.float32)
    b1r = b1.reshape(1, HH)
    b2r = b2.reshape(1, EMB)
    g1br = g1b.reshape(1, EMB)
    g2br = g2b.reshape(1, EMB)
    cbr = cb.reshape(1, CC)

    csrc_p, cdst_p = _sc_degree(src, dst, zeros_n, ones_c)
    hw1 = _tc_mlp(xp, W1, b1r, W2, b2r, g1W)
    hws1, dinv, cnt, gdd = _tc_prep(
        csrc_p[0].reshape(NPAD, 1), csrc_p[1].reshape(NPAD, 1),
        cdst_p[0].reshape(NPAD, 1), cdst_p[1].reshape(NPAD, 1), hw1)
    acc1 = _sc_conv(src, dst, hws1, zeros_nd)
    hws2, hw2 = _tc_mid(acc1[0], acc1[1], dinv, hw1, g1br, g2W)
    acc2 = _sc_conv(src, dst, hws2, zeros_nd)
    hn, base = _tc_final(acc2[0], acc2[1], dinv, hw2, g2br, cW, cbr, gdd)
    agg = _sc_sim(src, dst, hn, zeros_nd)
    score = _tc_score(agg[0], agg[1], hn, cnt, base)
    return score[:NN, 0]


# speedup vs baseline: 28.4031x; 1.1225x over previous
import functools

import jax
import jax.numpy as jnp
from jax import lax
from jax.experimental import pallas as pl
from jax.experimental.pallas import tpu as pltpu
from jax.experimental.pallas import tpu_sc as plsc

NN = 10000
EE = 320000
DD = 128
HH = 128
EMB = 64
CC = 2
AL, BE, GA = 0.6, 0.4, 0.1

NPAD = 10240
DUMP = NPAD - 1
NC = 2
NS = 16
NW = NC * NS
EPT = NPAD
CHUNK = 128
NCHUNK = EPT // CHUNK
EPAD = NW * EPT
RPT = NPAD // NS


def _leaky(z):
    return jnp.where(z >= 0, z, 0.01 * z)


def _mesh():
    return plsc.VectorSubcoreMesh(core_axis_name="core", subcore_axis_name="subcore")


_SC_PARAMS = pltpu.CompilerParams(use_tc_tiling_on_sc=False)


def _sc_degree(src3, dst3, zeros_n, ones_c):
    @functools.partial(
        pl.kernel,
        out_type=(
            jax.ShapeDtypeStruct((NC, NPAD), jnp.float32),
            jax.ShapeDtypeStruct((NC, NPAD), jnp.float32),
        ),
        mesh=_mesh(),
        compiler_params=_SC_PARAMS,
        scratch_types=[
            pltpu.VMEM((NCHUNK, CHUNK), jnp.int32),
            pltpu.VMEM((NCHUNK, CHUNK), jnp.int32),
            pltpu.VMEM((CHUNK,), jnp.float32),
            pltpu.VMEM_SHARED((NPAD,), jnp.float32),
            pltpu.VMEM_SHARED((NPAD,), jnp.float32),
            pltpu.SemaphoreType.DMA,
            pltpu.SemaphoreType.DMA,
        ],
    )
    def k(src_hbm, dst_hbm, z_hbm, ones_hbm, osrc_hbm, odst_hbm,
          src_v, dst_v, ones_v, csrc_sh, cdst_sh, sem_a, sem_b):
        cid = lax.axis_index("core")
        sid = lax.axis_index("subcore")
        wid = cid * NS + sid
        pltpu.sync_copy(src_hbm.at[wid], src_v)
        pltpu.sync_copy(dst_hbm.at[wid], dst_v)
        pltpu.sync_copy(ones_hbm, ones_v)
        r = pl.ds(sid * RPT, RPT)
        pltpu.sync_copy(z_hbm.at[r], csrc_sh.at[r])
        pltpu.sync_copy(z_hbm.at[r], cdst_sh.at[r])
        plsc.subcore_barrier()

        @pl.loop(0, NCHUNK)
        def _(j):
            pltpu.async_copy(ones_v, csrc_sh.at[src_v.at[j]], sem_a, add=True)
            pltpu.async_copy(ones_v, cdst_sh.at[dst_v.at[j]], sem_b, add=True)
            pltpu.make_async_copy(
                ones_v, csrc_sh.at[src_v.at[j]], sem_a).wait()
            pltpu.make_async_copy(
                ones_v, cdst_sh.at[dst_v.at[j]], sem_b).wait()

        plsc.subcore_barrier()
        pltpu.sync_copy(csrc_sh.at[r], osrc_hbm.at[cid, r])
        pltpu.sync_copy(cdst_sh.at[r], odst_hbm.at[cid, r])

    return k(src3, dst3, zeros_n, ones_c)


def _sc_conv(src3, dst3, table, zeros_nd):
    @functools.partial(
        pl.kernel,
        out_type=jax.ShapeDtypeStruct((NC, NPAD, EMB), jnp.float32),
        mesh=_mesh(),
        compiler_params=_SC_PARAMS,
        scratch_types=[
            pltpu.VMEM((NCHUNK, CHUNK), jnp.int32),
            pltpu.VMEM((NCHUNK, CHUNK), jnp.int32),
            pltpu.VMEM((4, CHUNK, EMB), jnp.float32),
            pltpu.VMEM_SHARED((NPAD, EMB), jnp.float32),
            pltpu.SemaphoreType.DMA,
            pltpu.SemaphoreType.DMA,
            pltpu.SemaphoreType.DMA,
            pltpu.SemaphoreType.DMA,
        ],
    )
    def k(src_hbm, dst_hbm, tab_hbm, z_hbm, out_hbm,
          src_v, dst_v, rows, acc_sh, s0, s1, s2, s3):
        cid = lax.axis_index("core")
        sid = lax.axis_index("subcore")
        wid = cid * NS + sid
        pltpu.sync_copy(src_hbm.at[wid], src_v)
        pltpu.sync_copy(dst_hbm.at[wid], dst_v)
        r = pl.ds(sid * RPT, RPT)
        pltpu.sync_copy(z_hbm.at[r], acc_sh.at[r])
        plsc.subcore_barrier()

        sems = (s0, s1, s2, s3)
        for q in range(4):
            pltpu.async_copy(tab_hbm.at[src_v.at[q]], rows.at[q], sems[q])

        @pl.loop(0, NCHUNK, step=4)
        def _(j):
            for q in range(4):
                pltpu.make_async_copy(
                    tab_hbm.at[src_v.at[0]], rows.at[q], sems[q]).wait()
                pltpu.sync_copy(rows.at[q], acc_sh.at[dst_v.at[j + q]],
                                add=True)

                @pl.when(j + q + 4 < NCHUNK)
                def _():
                    pltpu.async_copy(tab_hbm.at[src_v.at[j + q + 4]],
                                     rows.at[q], sems[q])

        plsc.subcore_barrier()
        pltpu.sync_copy(acc_sh.at[r], out_hbm.at[cid, r])

    return k(src3, dst3, table, zeros_nd)


def _sc_sim(src3, dst3, hn, zeros_nd):
    @functools.partial(
        pl.kernel,
        out_type=jax.ShapeDtypeStruct((NC, NPAD, EMB), jnp.float32),
        mesh=_mesh(),
        compiler_params=_SC_PARAMS,
        scratch_types=[
            pltpu.VMEM((NCHUNK, CHUNK), jnp.int32),
            pltpu.VMEM((NCHUNK, CHUNK), jnp.int32),
            pltpu.VMEM((4, CHUNK, EMB), jnp.float32),
            pltpu.VMEM((4, CHUNK, EMB), jnp.float32),
            pltpu.VMEM_SHARED((NPAD, EMB), jnp.float32),
            pltpu.SemaphoreType.DMA,
            pltpu.SemaphoreType.DMA,
            pltpu.SemaphoreType.DMA,
            pltpu.SemaphoreType.DMA,
            pltpu.SemaphoreType.DMA,
            pltpu.SemaphoreType.DMA,
            pltpu.SemaphoreType.DMA,
            pltpu.SemaphoreType.DMA,
        ],
    )
    def k(src_hbm, dst_hbm, hn_hbm, z_hbm, out_hbm,
          src_v, dst_v, rows_s, rows_t, agg_sh,
          ss0, ss1, ss2, ss3, ts0, ts1, ts2, ts3):
        cid = lax.axis_index("core")
        sid = lax.axis_index("subcore")
        wid = cid * NS + sid
        pltpu.sync_copy(src_hbm.at[wid], src_v)
        pltpu.sync_copy(dst_hbm.at[wid], dst_v)
        r = pl.ds(sid * RPT, RPT)
        pltpu.sync_copy(z_hbm.at[r], agg_sh.at[r])
        plsc.subcore_barrier()

        ssem = (ss0, ss1, ss2, ss3)
        tsem = (ts0, ts1, ts2, ts3)
        for q in range(4):
            pltpu.async_copy(hn_hbm.at[src_v.at[q]], rows_s.at[q], ssem[q])
            pltpu.async_copy(hn_hbm.at[dst_v.at[q]], rows_t.at[q], tsem[q])

        @pl.loop(0, NCHUNK, step=4)
        def _(j):
            for q in range(4):
                pltpu.make_async_copy(
                    hn_hbm.at[src_v.at[0]], rows_s.at[q], ssem[q]).wait()
                pltpu.make_async_copy(
                    hn_hbm.at[dst_v.at[0]], rows_t.at[q], tsem[q]).wait()
                pltpu.sync_copy(rows_s.at[q], agg_sh.at[dst_v.at[j + q]],
                                add=True)
                pltpu.sync_copy(rows_t.at[q], agg_sh.at[src_v.at[j + q]],
                                add=True)

                @pl.when(j + q + 4 < NCHUNK)
                def _():
                    pltpu.async_copy(hn_hbm.at[src_v.at[j + q + 4]],
                                     rows_s.at[q], ssem[q])
                    pltpu.async_copy(hn_hbm.at[dst_v.at[j + q + 4]],
                                     rows_t.at[q], tsem[q])

        plsc.subcore_barrier()
        pltpu.sync_copy(agg_sh.at[r], out_hbm.at[cid, r])

    return k(src3, dst3, hn, zeros_nd)


def _tc_mlp(xp, W1, b1r, W2, b2r, g1W):
    def body(x_ref, w1_ref, b1_ref, w2_ref, b2_ref, g1_ref, hw1_ref):
        h = _leaky(jnp.dot(x_ref[...], w1_ref[...],
                           preferred_element_type=jnp.float32) + b1_ref[...])
        h2 = _leaky(jnp.dot(h, w2_ref[...],
                            preferred_element_type=jnp.float32) + b2_ref[...])
        hw1_ref[...] = jnp.dot(h2, g1_ref[...],
                               preferred_element_type=jnp.float32)

    return pl.pallas_call(
        body,
        out_shape=jax.ShapeDtypeStruct((NPAD, EMB), jnp.float32),
    )(xp, W1, b1r, W2, b2r, g1W)


def _tc_prep(csrc0, csrc1, cdst0, cdst1, hw1):
    def body(cs0, cs1, cd0, cd1, hw1_ref, hws1_ref, dinv_ref, cnt_ref, gdd_ref):
        cdst = cd0[...] + cd1[...]
        cnt = cs0[...] + cs1[...] + cdst
        deg = cdst + 1.0
        dinv = lax.rsqrt(deg)
        hws1_ref[...] = dinv * hw1_ref[...]
        dinv_ref[...] = dinv
        cnt_ref[...] = cnt
        mask = (lax.broadcasted_iota(jnp.int32, (NPAD, 1), 0) < NN).astype(
            jnp.float32)
        cm = jnp.sum(cnt * mask) / NN
        cs = jnp.sqrt(jnp.sum((cnt - cm) ** 2 * mask) / (NN - 1))
        gdd_ref[...] = GA * (cnt - cm) / (cs + 1e-8)

    return pl.pallas_call(
        body,
        out_shape=(
            jax.ShapeDtypeStruct((NPAD, EMB), jnp.float32),
            jax.ShapeDtypeStruct((NPAD, 1), jnp.float32),
            jax.ShapeDtypeStruct((NPAD, 1), jnp.float32),
            jax.ShapeDtypeStruct((NPAD, 1), jnp.float32),
        ),
    )(csrc0, csrc1, cdst0, cdst1, hw1)


def _tc_mid(acc0, acc1, dinv, hw1, g1br, g2W):
    def body(a0, a1, dinv_ref, hw1_ref, g1b_ref, g2w_ref, hws2_ref, hw2_ref):
        dinv = dinv_ref[...]
        h3 = _leaky(dinv * (a0[...] + a1[...])
                    + dinv * dinv * hw1_ref[...] + g1b_ref[...])
        hw2 = jnp.dot(h3, g2w_ref[...], preferred_element_type=jnp.float32)
        hw2_ref[...] = hw2
        hws2_ref[...] = dinv * hw2

    return pl.pallas_call(
        body,
        out_shape=(
            jax.ShapeDtypeStruct((NPAD, EMB), jnp.float32),
            jax.ShapeDtypeStruct((NPAD, EMB), jnp.float32),
        ),
    )(acc0, acc1, dinv, hw1, g1br, g2W)


def _tc_final(acc0, acc1, dinv, hw2, g2br, cW, cbr, gdd):
    def body(a0, a1, dinv_ref, hw2_ref, g2b_ref, cw_ref, cb_ref, gdd_ref,
             hn_ref, base_ref):
        dinv = dinv_ref[...]
        h4 = _leaky(dinv * (a0[...] + a1[...])
                    + dinv * dinv * hw2_ref[...] + g2b_ref[...])
        logits = jnp.dot(h4, cw_ref[...],
                         preferred_element_type=jnp.float32) + cb_ref[...]
        m = jnp.max(logits, axis=1, keepdims=True)
        z = logits - m
        lse = jnp.log(jnp.sum(jnp.exp(z), axis=1, keepdims=True))
        logp = z - lse
        p = jnp.exp(logp)
        ent = -jnp.sum(p * logp, axis=1, keepdims=True)
        mask = (lax.broadcasted_iota(jnp.int32, (NPAD, 1), 0) < NN).astype(
            jnp.float32)
        em = jnp.sum(ent * mask) / NN
        es = jnp.sqrt(jnp.sum((ent - em) ** 2 * mask) / (NN - 1))
        std_ent = (ent - em) / (es + 1e-8)
        nrm = jnp.maximum(
            jnp.sqrt(jnp.sum(h4 * h4, axis=1, keepdims=True)), 1e-8)
        hn_ref[...] = h4 / nrm
        base_ref[...] = AL * std_ent + BE + gdd_ref[...]

    return pl.pallas_call(
        body,
        out_shape=(
            jax.ShapeDtypeStruct((NPAD, EMB), jnp.float32),
            jax.ShapeDtypeStruct((NPAD, 1), jnp.float32),
        ),
    )(acc0, acc1, dinv, hw2, g2br, cW, cbr, gdd)


def _tc_score(agg0, agg1, hn, cnt, base):
    def body(a0, a1, hn_ref, cnt_ref, base_ref, out_ref):
        ssum = jnp.sum(hn_ref[...] * (a0[...] + a1[...]), axis=1,
                       keepdims=True)
        cnt = cnt_ref[...]
        avg = jnp.where(cnt > 0, ssum / jnp.maximum(cnt, 1.0), 1.0)
        out_ref[...] = base_ref[...] - BE * avg

    return pl.pallas_call(
        body,
        out_shape=jax.ShapeDtypeStruct((NPAD, 1), jnp.float32),
    )(agg0, agg1, hn, cnt, base)


def kernel(x, edge_index, W1, b1, W2, b2, g1W, g1b, g2W, g2b, cW, cb):
    xp = jnp.pad(x, ((0, NPAD - NN), (0, 0)))
    pad_idx = NN + jnp.arange(EPAD - EE, dtype=jnp.int32) % (NPAD - NN)
    src = jnp.concatenate([edge_index[0], pad_idx]).reshape(NW, NCHUNK, CHUNK)
    dst = jnp.concatenate([edge_index[1], pad_idx]).reshape(NW, NCHUNK, CHUNK)
    zeros_n = jnp.zeros((NPAD,), jnp.float32)
    zeros_nd = jnp.zeros((NPAD, EMB), jnp.float32)
    ones_c = jnp.ones((CHUNK,), jnp.float32)
    b1r = b1.reshape(1, HH)
    b2r = b2.reshape(1, EMB)
    g1br = g1b.reshape(1, EMB)
    g2br = g2b.reshape(1, EMB)
    cbr = cb.reshape(1, CC)

    csrc_p, cdst_p = _sc_degree(src, dst, zeros_n, ones_c)
    hw1 = _tc_mlp(xp, W1, b1r, W2, b2r, g1W)
    hws1, dinv, cnt, gdd = _tc_prep(
        csrc_p[0].reshape(NPAD, 1), csrc_p[1].reshape(NPAD, 1),
        cdst_p[0].reshape(NPAD, 1), cdst_p[1].reshape(NPAD, 1), hw1)
    acc1 = _sc_conv(src, dst, hws1, zeros_nd)
    hws2, hw2 = _tc_mid(acc1[0], acc1[1], dinv, hw1, g1br, g2W)
    acc2 = _sc_conv(src, dst, hws2, zeros_nd)
    hn, base = _tc_final(acc2[0], acc2[1], dinv, hw2, g2br, cW, cbr, gdd)
    agg = _sc_sim(src, dst, hn, zeros_nd)
    score = _tc_score(agg[0], agg[1], hn, cnt, base)
    return score[:NN, 0]


# speedup vs baseline: 31.8238x; 1.1204x over previous
import functools

import jax
import jax.numpy as jnp
from jax import lax
from jax.experimental import pallas as pl
from jax.experimental.pallas import tpu as pltpu
from jax.experimental.pallas import tpu_sc as plsc

NN = 10000
EE = 320000
DD = 128
HH = 128
EMB = 64
CC = 2
AL, BE, GA = 0.6, 0.4, 0.1

NPAD = 10240
DUMP = NPAD - 1
NC = 2
NS = 16
NW = NC * NS
EPT = NPAD
CHUNK = 128
NCHUNK = EPT // CHUNK
EPAD = NW * EPT
RPT = NPAD // NS


def _leaky(z):
    return jnp.where(z >= 0, z, 0.01 * z)


def _mesh():
    return plsc.VectorSubcoreMesh(core_axis_name="core", subcore_axis_name="subcore")


_SC_PARAMS = pltpu.CompilerParams(use_tc_tiling_on_sc=False)


def _sc_degree(src3, dst3, zeros_n, ones_c):
    @functools.partial(
        pl.kernel,
        out_type=(
            jax.ShapeDtypeStruct((NC, NPAD), jnp.float32),
            jax.ShapeDtypeStruct((NC, NPAD), jnp.float32),
        ),
        mesh=_mesh(),
        compiler_params=_SC_PARAMS,
        scratch_types=[
            pltpu.VMEM((NCHUNK, CHUNK), jnp.int32),
            pltpu.VMEM((NCHUNK, CHUNK), jnp.int32),
            pltpu.VMEM((CHUNK,), jnp.float32),
            pltpu.VMEM_SHARED((NPAD,), jnp.float32),
            pltpu.VMEM_SHARED((NPAD,), jnp.float32),
            pltpu.SemaphoreType.DMA,
            pltpu.SemaphoreType.DMA,
        ],
    )
    def k(src_hbm, dst_hbm, z_hbm, ones_hbm, osrc_hbm, odst_hbm,
          src_v, dst_v, ones_v, csrc_sh, cdst_sh, sem_a, sem_b):
        cid = lax.axis_index("core")
        sid = lax.axis_index("subcore")
        wid = cid * NS + sid
        pltpu.sync_copy(src_hbm.at[wid], src_v)
        pltpu.sync_copy(dst_hbm.at[wid], dst_v)
        pltpu.sync_copy(ones_hbm, ones_v)
        r = pl.ds(sid * RPT, RPT)
        pltpu.sync_copy(z_hbm.at[r], csrc_sh.at[r])
        pltpu.sync_copy(z_hbm.at[r], cdst_sh.at[r])
        plsc.subcore_barrier()

        @pl.loop(0, NCHUNK)
        def _(j):
            pltpu.async_copy(ones_v, csrc_sh.at[src_v.at[j]], sem_a, add=True)
            pltpu.async_copy(ones_v, cdst_sh.at[dst_v.at[j]], sem_b, add=True)
            pltpu.make_async_copy(
                ones_v, csrc_sh.at[src_v.at[j]], sem_a).wait()
            pltpu.make_async_copy(
                ones_v, cdst_sh.at[dst_v.at[j]], sem_b).wait()

        plsc.subcore_barrier()
        pltpu.sync_copy(csrc_sh.at[r], osrc_hbm.at[cid, r])
        pltpu.sync_copy(cdst_sh.at[r], odst_hbm.at[cid, r])

    return k(src3, dst3, zeros_n, ones_c)


def _sc_conv(src3, dst3, table, zeros_nd):
    @functools.partial(
        pl.kernel,
        out_type=jax.ShapeDtypeStruct((NC, NPAD, EMB), jnp.float32),
        mesh=_mesh(),
        compiler_params=_SC_PARAMS,
        scratch_types=[
            pltpu.VMEM((NCHUNK, CHUNK), jnp.int32),
            pltpu.VMEM((NCHUNK, CHUNK), jnp.int32),
            pltpu.VMEM((4, CHUNK, EMB), jnp.float32),
            pltpu.VMEM_SHARED((NPAD, EMB), jnp.float32),
            pltpu.SemaphoreType.DMA,
            pltpu.SemaphoreType.DMA,
            pltpu.SemaphoreType.DMA,
            pltpu.SemaphoreType.DMA,
        ],
    )
    def k(src_hbm, dst_hbm, tab_hbm, z_hbm, out_hbm,
          src_v, dst_v, rows, acc_sh, s0, s1, s2, s3):
        cid = lax.axis_index("core")
        sid = lax.axis_index("subcore")
        wid = cid * NS + sid
        pltpu.sync_copy(src_hbm.at[wid], src_v)
        pltpu.sync_copy(dst_hbm.at[wid], dst_v)
        r = pl.ds(sid * RPT, RPT)
        pltpu.sync_copy(z_hbm.at[r], acc_sh.at[r])
        plsc.subcore_barrier()

        sems = (s0, s1, s2, s3)
        for q in range(4):
            pltpu.async_copy(tab_hbm.at[src_v.at[q]], rows.at[q], sems[q])

        @pl.loop(0, NCHUNK, step=4)
        def _(j):
            for q in range(4):
                pltpu.make_async_copy(
                    tab_hbm.at[src_v.at[0]], rows.at[q], sems[q]).wait()
                pltpu.sync_copy(rows.at[q], acc_sh.at[dst_v.at[j + q]],
                                add=True)

                @pl.when(j + q + 4 < NCHUNK)
                def _():
                    pltpu.async_copy(tab_hbm.at[src_v.at[j + q + 4]],
                                     rows.at[q], sems[q])

        plsc.subcore_barrier()
        pltpu.sync_copy(acc_sh.at[r], out_hbm.at[cid, r])

    return k(src3, dst3, table, zeros_nd)


def _sc_sim(src3, dst3, hn, zeros_nd):
    @functools.partial(
        pl.kernel,
        out_type=jax.ShapeDtypeStruct((NC, NPAD, EMB), jnp.float32),
        mesh=_mesh(),
        compiler_params=_SC_PARAMS,
        scratch_types=[
            pltpu.VMEM((NCHUNK, CHUNK), jnp.int32),
            pltpu.VMEM((NCHUNK, CHUNK), jnp.int32),
            pltpu.VMEM((4, CHUNK, EMB), jnp.float32),
            pltpu.VMEM((4, CHUNK, EMB), jnp.float32),
            pltpu.VMEM_SHARED((NPAD, EMB), jnp.float32),
            pltpu.SemaphoreType.DMA,
            pltpu.SemaphoreType.DMA,
            pltpu.SemaphoreType.DMA,
            pltpu.SemaphoreType.DMA,
            pltpu.SemaphoreType.DMA,
            pltpu.SemaphoreType.DMA,
            pltpu.SemaphoreType.DMA,
            pltpu.SemaphoreType.DMA,
        ],
    )
    def k(src_hbm, dst_hbm, hn_hbm, z_hbm, out_hbm,
          src_v, dst_v, rows_s, rows_t, agg_sh,
          ss0, ss1, ss2, ss3, ts0, ts1, ts2, ts3):
        cid = lax.axis_index("core")
        sid = lax.axis_index("subcore")
        wid = cid * NS + sid
        pltpu.sync_copy(src_hbm.at[wid], src_v)
        pltpu.sync_copy(dst_hbm.at[wid], dst_v)
        r = pl.ds(sid * RPT, RPT)
        pltpu.sync_copy(z_hbm.at[r], agg_sh.at[r])
        plsc.subcore_barrier()

        ssem = (ss0, ss1, ss2, ss3)
        tsem = (ts0, ts1, ts2, ts3)
        for q in range(4):
            pltpu.async_copy(hn_hbm.at[src_v.at[q]], rows_s.at[q], ssem[q])
            pltpu.async_copy(hn_hbm.at[dst_v.at[q]], rows_t.at[q], tsem[q])

        @pl.loop(0, NCHUNK, step=4)
        def _(j):
            for q in range(4):
                pltpu.make_async_copy(
                    hn_hbm.at[src_v.at[0]], rows_s.at[q], ssem[q]).wait()
                pltpu.make_async_copy(
                    hn_hbm.at[dst_v.at[0]], rows_t.at[q], tsem[q]).wait()
                pltpu.sync_copy(rows_s.at[q], agg_sh.at[dst_v.at[j + q]],
                                add=True)
                pltpu.sync_copy(rows_t.at[q], agg_sh.at[src_v.at[j + q]],
                                add=True)

                @pl.when(j + q + 4 < NCHUNK)
                def _():
                    pltpu.async_copy(hn_hbm.at[src_v.at[j + q + 4]],
                                     rows_s.at[q], ssem[q])
                    pltpu.async_copy(hn_hbm.at[dst_v.at[j + q + 4]],
                                     rows_t.at[q], tsem[q])

        plsc.subcore_barrier()
        pltpu.sync_copy(agg_sh.at[r], out_hbm.at[cid, r])

    return k(src3, dst3, hn, zeros_nd)


RB = 1280
GRID = NPAD // RB

_ROWS_E = pl.BlockSpec((RB, EMB), lambda i: (i, 0))
_ROWS_1 = pl.BlockSpec((RB, 1), lambda i: (i, 0))
_ACC = pl.BlockSpec((NC, RB, EMB), lambda i: (0, i, 0))


def _full(shape):
    return pl.BlockSpec(shape, lambda i: tuple(0 for _ in shape))


def _tc_mlp(xp, W1, b1r, W2, b2r, g1W):
    def body(x_ref, w1_ref, b1_ref, w2_ref, b2_ref, g1_ref, hw1_ref):
        h = _leaky(jnp.dot(x_ref[...], w1_ref[...],
                           preferred_element_type=jnp.float32) + b1_ref[...])
        h2 = _leaky(jnp.dot(h, w2_ref[...],
                            preferred_element_type=jnp.float32) + b2_ref[...])
        hw1_ref[...] = jnp.dot(h2, g1_ref[...],
                               preferred_element_type=jnp.float32)

    return pl.pallas_call(
        body,
        grid=(GRID,),
        in_specs=[pl.BlockSpec((RB, DD), lambda i: (i, 0)),
                  _full((DD, HH)), _full((1, HH)),
                  _full((HH, EMB)), _full((1, EMB)),
                  _full((EMB, EMB))],
        out_specs=_ROWS_E,
        out_shape=jax.ShapeDtypeStruct((NPAD, EMB), jnp.float32),
    )(xp, W1, b1r, W2, b2r, g1W)


def _tc_prep(degc, hw1):
    def body(deg_ref, hw1_ref, hws1_ref, dinv_ref):
        dinv = lax.rsqrt(deg_ref[...] + 1.0)
        hws1_ref[...] = dinv * hw1_ref[...]
        dinv_ref[...] = dinv

    return pl.pallas_call(
        body,
        grid=(GRID,),
        in_specs=[_ROWS_1, _ROWS_E],
        out_specs=(_ROWS_E, _ROWS_1),
        out_shape=(
            jax.ShapeDtypeStruct((NPAD, EMB), jnp.float32),
            jax.ShapeDtypeStruct((NPAD, 1), jnp.float32),
        ),
    )(degc, hw1)


def _tc_mid(acc, dinv, hw1, g1br, g2W):
    def body(a_ref, dinv_ref, hw1_ref, g1b_ref, g2w_ref, hws2_ref, hw2_ref):
        dinv = dinv_ref[...]
        h3 = _leaky(dinv * (a_ref[0] + a_ref[1])
                    + dinv * dinv * hw1_ref[...] + g1b_ref[...])
        hw2 = jnp.dot(h3, g2w_ref[...], preferred_element_type=jnp.float32)
        hw2_ref[...] = hw2
        hws2_ref[...] = dinv * hw2

    return pl.pallas_call(
        body,
        grid=(GRID,),
        in_specs=[_ACC, _ROWS_1, _ROWS_E, _full((1, EMB)),
                  _full((EMB, EMB))],
        out_specs=(_ROWS_E, _ROWS_E),
        out_shape=(
            jax.ShapeDtypeStruct((NPAD, EMB), jnp.float32),
            jax.ShapeDtypeStruct((NPAD, EMB), jnp.float32),
        ),
    )(acc, dinv, hw1, g1br, g2W)


def _tc_final(acc, dinv, hw2, g2br, cW, cbr):
    def body(a_ref, dinv_ref, hw2_ref, g2b_ref, cw_ref, cb_ref,
             hn_ref, ent_ref):
        dinv = dinv_ref[...]
        h4 = _leaky(dinv * (a_ref[0] + a_ref[1])
                    + dinv * dinv * hw2_ref[...] + g2b_ref[...])
        logits = jnp.dot(h4, cw_ref[...],
                         preferred_element_type=jnp.float32) + cb_ref[...]
        m = jnp.max(logits, axis=1, keepdims=True)
        z = logits - m
        lse = jnp.log(jnp.sum(jnp.exp(z), axis=1, keepdims=True))
        logp = z - lse
        p = jnp.exp(logp)
        ent_ref[...] = -jnp.sum(p * logp, axis=1, keepdims=True)
        nrm = jnp.maximum(
            jnp.sqrt(jnp.sum(h4 * h4, axis=1, keepdims=True)), 1e-8)
        hn_ref[...] = h4 / nrm

    return pl.pallas_call(
        body,
        grid=(GRID,),
        in_specs=[_ACC, _ROWS_1, _ROWS_E, _full((1, EMB)),
                  _full((EMB, CC)), _full((1, CC))],
        out_specs=(_ROWS_E, _ROWS_1),
        out_shape=(
            jax.ShapeDtypeStruct((NPAD, EMB), jnp.float32),
            jax.ShapeDtypeStruct((NPAD, 1), jnp.float32),
        ),
    )(acc, dinv, hw2, g2br, cW, cbr)


def _tc_score(agg, hn, ent, cnt):
    def body(a_ref, hn_ref, ent_ref, cnt_ref, out_ref):
        mask = (lax.broadcasted_iota(jnp.int32, (NPAD, 1), 0) < NN).astype(
            jnp.float32)
        ent = ent_ref[...]
        em = jnp.sum(ent * mask) / NN
        es = jnp.sqrt(jnp.sum((ent - em) ** 2 * mask) / (NN - 1))
        std_ent = (ent - em) / (es + 1e-8)
        cnt = cnt_ref[...]
        cm = jnp.sum(cnt * mask) / NN
        cs = jnp.sqrt(jnp.sum((cnt - cm) ** 2 * mask) / (NN - 1))
        gdd = GA * (cnt - cm) / (cs + 1e-8)
        ssum = jnp.sum(hn_ref[...] * (a_ref[0] + a_ref[1]), axis=1,
                       keepdims=True)
        avg = jnp.where(cnt > 0, ssum / jnp.maximum(cnt, 1.0), 1.0)
        out_ref[...] = AL * std_ent + BE * (1.0 - avg) + gdd

    return pl.pallas_call(
        body,
        out_shape=jax.ShapeDtypeStruct((NPAD, 1), jnp.float32),
    )(agg, hn, ent, cnt)


def kernel(x, edge_index, W1, b1, W2, b2, g1W, g1b, g2W, g2b, cW, cb):
    xp = jnp.pad(x, ((0, NPAD - NN), (0, 0)))
    pad_idx = NN + jnp.arange(EPAD - EE, dtype=jnp.int32) % (NPAD - NN)
    src = jnp.concatenate([edge_index[0], pad_idx]).reshape(NW, NCHUNK, CHUNK)
    dst = jnp.concatenate([edge_index[1], pad_idx]).reshape(NW, NCHUNK, CHUNK)
    zeros_n = jnp.zeros((NPAD,), jnp.float32)
    zeros_nd = jnp.zeros((NPAD, EMB), jnp.float32)
    ones_c = jnp.ones((CHUNK,), jnp.float32)
    b1r = b1.reshape(1, HH)
    b2r = b2.reshape(1, EMB)
    g1br = g1b.reshape(1, EMB)
    g2br = g2b.reshape(1, EMB)
    cbr = cb.reshape(1, CC)

    csrc_p, cdst_p = _sc_degree(src, dst, zeros_n, ones_c)
    hw1 = _tc_mlp(xp, W1, b1r, W2, b2r, g1W)
    degc = (cdst_p[0] + cdst_p[1]).reshape(NPAD, 1)
    cntc = degc + (csrc_p[0] + csrc_p[1]).reshape(NPAD, 1)
    hws1, dinv = _tc_prep(degc, hw1)
    acc1 = _sc_conv(src, dst, hws1, zeros_nd)
    hws2, hw2 = _tc_mid(acc1, dinv, hw1, g1br, g2W)
    acc2 = _sc_conv(src, dst, hws2, zeros_nd)
    hn, ent = _tc_final(acc2, dinv, hw2, g2br, cW, cbr)
    agg = _sc_sim(src, dst, hn, zeros_nd)
    score = _tc_score(agg, hn, ent, cntc)
    return score[:NN, 0]


# speedup vs baseline: 34.3735x; 1.0801x over previous
import functools

import jax
import jax.numpy as jnp
from jax import lax
from jax.experimental import pallas as pl
from jax.experimental.pallas import tpu as pltpu
from jax.experimental.pallas import tpu_sc as plsc

NN = 10000
EE = 320000
DD = 128
HH = 128
EMB = 64
CC = 2
AL, BE, GA = 0.6, 0.4, 0.1

NPAD = 10240
DUMP = NPAD - 1
NC = 2
NS = 16
NW = NC * NS
EPT = NPAD
CHUNK = 128
NCHUNK = EPT // CHUNK
EPAD = NW * EPT
RPT = NPAD // NS


def _leaky(z):
    return jnp.where(z >= 0, z, 0.01 * z)


def _mesh():
    return plsc.VectorSubcoreMesh(core_axis_name="core", subcore_axis_name="subcore")


_SC_PARAMS = pltpu.CompilerParams(use_tc_tiling_on_sc=False)


def _sc_degree(src3, dst3, zeros_n, ones_c):
    @functools.partial(
        pl.kernel,
        out_type=(
            jax.ShapeDtypeStruct((NC, NPAD), jnp.float32),
            jax.ShapeDtypeStruct((NC, NPAD), jnp.float32),
        ),
        mesh=_mesh(),
        compiler_params=_SC_PARAMS,
        scratch_types=[
            pltpu.VMEM((NCHUNK, CHUNK), jnp.int32),
            pltpu.VMEM((NCHUNK, CHUNK), jnp.int32),
            pltpu.VMEM((CHUNK,), jnp.float32),
            pltpu.VMEM_SHARED((NPAD,), jnp.float32),
            pltpu.VMEM_SHARED((NPAD,), jnp.float32),
            pltpu.SemaphoreType.DMA,
            pltpu.SemaphoreType.DMA,
        ],
    )
    def k(src_hbm, dst_hbm, z_hbm, ones_hbm, osrc_hbm, odst_hbm,
          src_v, dst_v, ones_v, csrc_sh, cdst_sh, sem_a, sem_b):
        cid = lax.axis_index("core")
        sid = lax.axis_index("subcore")
        wid = cid * NS + sid
        pltpu.sync_copy(src_hbm.at[wid], src_v)
        pltpu.sync_copy(dst_hbm.at[wid], dst_v)
        pltpu.sync_copy(ones_hbm, ones_v)
        r = pl.ds(sid * RPT, RPT)
        pltpu.sync_copy(z_hbm.at[r], csrc_sh.at[r])
        pltpu.sync_copy(z_hbm.at[r], cdst_sh.at[r])
        plsc.subcore_barrier()

        @pl.loop(0, NCHUNK)
        def _(j):
            pltpu.async_copy(ones_v, csrc_sh.at[src_v.at[j]], sem_a, add=True)
            pltpu.async_copy(ones_v, cdst_sh.at[dst_v.at[j]], sem_b, add=True)
            pltpu.make_async_copy(
                ones_v, csrc_sh.at[src_v.at[j]], sem_a).wait()
            pltpu.make_async_copy(
                ones_v, cdst_sh.at[dst_v.at[j]], sem_b).wait()

        plsc.subcore_barrier()
        pltpu.sync_copy(csrc_sh.at[r], osrc_hbm.at[cid, r])
        pltpu.sync_copy(cdst_sh.at[r], odst_hbm.at[cid, r])

    return k(src3, dst3, zeros_n, ones_c)


def _sc_conv(src3, dst3, table, zeros_nd):
    @functools.partial(
        pl.kernel,
        out_type=jax.ShapeDtypeStruct((NC, NPAD, 2 * EMB), jnp.float32),
        mesh=_mesh(),
        compiler_params=_SC_PARAMS,
        scratch_types=[
            pltpu.VMEM((NCHUNK, CHUNK), jnp.int32),
            pltpu.VMEM((NCHUNK, CHUNK), jnp.int32),
            pltpu.VMEM((4, CHUNK, EMB), jnp.float32),
            pltpu.VMEM_SHARED((NPAD, EMB), jnp.float32),
            pltpu.SemaphoreType.DMA,
            pltpu.SemaphoreType.DMA,
            pltpu.SemaphoreType.DMA,
            pltpu.SemaphoreType.DMA,
        ],
    )
    def k(src_hbm, dst_hbm, tab_hbm, z_hbm, out_hbm,
          src_v, dst_v, rows, acc_sh, s0, s1, s2, s3):
        cid = lax.axis_index("core")
        sid = lax.axis_index("subcore")
        wid = cid * NS + sid
        pltpu.sync_copy(src_hbm.at[wid], src_v)
        pltpu.sync_copy(dst_hbm.at[wid], dst_v)
        r = pl.ds(sid * RPT, RPT)
        pltpu.sync_copy(z_hbm.at[r], acc_sh.at[r])
        plsc.subcore_barrier()

        sems = (s0, s1, s2, s3)
        for q in range(4):
            pltpu.async_copy(tab_hbm.at[src_v.at[q]], rows.at[q], sems[q])

        @pl.loop(0, NCHUNK, step=4)
        def _(j):
            for q in range(4):
                pltpu.make_async_copy(
                    tab_hbm.at[src_v.at[0]], rows.at[q], sems[q]).wait()
                pltpu.sync_copy(rows.at[q], acc_sh.at[dst_v.at[j + q]],
                                add=True)

                @pl.when(j + q + 4 < NCHUNK)
                def _():
                    pltpu.async_copy(tab_hbm.at[src_v.at[j + q + 4]],
                                     rows.at[q], sems[q])

        plsc.subcore_barrier()
        pltpu.sync_copy(acc_sh.at[r], out_hbm.at[cid, r, pl.ds(0, EMB)])

    return k(src3, dst3, table, zeros_nd)


def _sc_sim(src3, dst3, hn, zeros_nd):
    @functools.partial(
        pl.kernel,
        out_type=jax.ShapeDtypeStruct((NC, NPAD, 2 * EMB), jnp.float32),
        mesh=_mesh(),
        compiler_params=_SC_PARAMS,
        scratch_types=[
            pltpu.VMEM((NCHUNK, CHUNK), jnp.int32),
            pltpu.VMEM((NCHUNK, CHUNK), jnp.int32),
            pltpu.VMEM((4, CHUNK, EMB), jnp.float32),
            pltpu.VMEM((4, CHUNK, EMB), jnp.float32),
            pltpu.VMEM_SHARED((NPAD, EMB), jnp.float32),
            pltpu.SemaphoreType.DMA,
            pltpu.SemaphoreType.DMA,
            pltpu.SemaphoreType.DMA,
            pltpu.SemaphoreType.DMA,
            pltpu.SemaphoreType.DMA,
            pltpu.SemaphoreType.DMA,
            pltpu.SemaphoreType.DMA,
            pltpu.SemaphoreType.DMA,
        ],
    )
    def k(src_hbm, dst_hbm, hn_hbm, z_hbm, out_hbm,
          src_v, dst_v, rows_s, rows_t, agg_sh,
          ss0, ss1, ss2, ss3, ts0, ts1, ts2, ts3):
        cid = lax.axis_index("core")
        sid = lax.axis_index("subcore")
        wid = cid * NS + sid
        pltpu.sync_copy(src_hbm.at[wid], src_v)
        pltpu.sync_copy(dst_hbm.at[wid], dst_v)
        r = pl.ds(sid * RPT, RPT)
        pltpu.sync_copy(z_hbm.at[r], agg_sh.at[r])
        plsc.subcore_barrier()

        ssem = (ss0, ss1, ss2, ss3)
        tsem = (ts0, ts1, ts2, ts3)
        for q in range(4):
            pltpu.async_copy(hn_hbm.at[src_v.at[q]], rows_s.at[q], ssem[q])
            pltpu.async_copy(hn_hbm.at[dst_v.at[q]], rows_t.at[q], tsem[q])

        @pl.loop(0, NCHUNK, step=4)
        def _(j):
            for q in range(4):
                pltpu.make_async_copy(
                    hn_hbm.at[src_v.at[0]], rows_s.at[q], ssem[q]).wait()
                pltpu.make_async_copy(
                    hn_hbm.at[dst_v.at[0]], rows_t.at[q], tsem[q]).wait()
                pltpu.sync_copy(rows_s.at[q], agg_sh.at[dst_v.at[j + q]],
                                add=True)
                pltpu.sync_copy(rows_t.at[q], agg_sh.at[src_v.at[j + q]],
                                add=True)

                @pl.when(j + q + 4 < NCHUNK)
                def _():
                    pltpu.async_copy(hn_hbm.at[src_v.at[j + q + 4]],
                                     rows_s.at[q], ssem[q])
                    pltpu.async_copy(hn_hbm.at[dst_v.at[j + q + 4]],
                                     rows_t.at[q], tsem[q])

        plsc.subcore_barrier()
        pltpu.sync_copy(agg_sh.at[r], out_hbm.at[cid, r, pl.ds(0, EMB)])

    return k(src3, dst3, hn, zeros_nd)


RB = 1280
GRID = NPAD // RB

_ROWS_E = pl.BlockSpec((RB, EMB), lambda i: (i, 0))
_ROWS_1 = pl.BlockSpec((RB, 1), lambda i: (i, 0))
_ACC = pl.BlockSpec((NC, RB, 2 * EMB), lambda i: (0, i, 0))


def _full(shape):
    return pl.BlockSpec(shape, lambda i: tuple(0 for _ in shape))


def _tc_mlp(xp, W1, b1r, W2, b2r, g1W):
    def body(x_ref, w1_ref, b1_ref, w2_ref, b2_ref, g1_ref, hw1_ref):
        h = _leaky(jnp.dot(x_ref[...], w1_ref[...],
                           preferred_element_type=jnp.float32) + b1_ref[...])
        h2 = _leaky(jnp.dot(h, w2_ref[...],
                            preferred_element_type=jnp.float32) + b2_ref[...])
        hw1_ref[...] = jnp.dot(h2, g1_ref[...],
                               preferred_element_type=jnp.float32)

    return pl.pallas_call(
        body,
        grid=(GRID,),
        in_specs=[pl.BlockSpec((RB, DD), lambda i: (i, 0)),
                  _full((DD, HH)), _full((1, HH)),
                  _full((HH, EMB)), _full((1, EMB)),
                  _full((EMB, EMB))],
        out_specs=_ROWS_E,
        out_shape=jax.ShapeDtypeStruct((NPAD, EMB), jnp.float32),
    )(xp, W1, b1r, W2, b2r, g1W)


def _tc_prep(degc, hw1):
    def body(deg_ref, hw1_ref, hws1_ref, dinv_ref):
        dinv = lax.rsqrt(deg_ref[...] + 1.0)
        hws1_ref[...] = dinv * hw1_ref[...]
        dinv_ref[...] = dinv

    return pl.pallas_call(
        body,
        grid=(GRID,),
        in_specs=[_ROWS_1, _ROWS_E],
        out_specs=(_ROWS_E, _ROWS_1),
        out_shape=(
            jax.ShapeDtypeStruct((NPAD, EMB), jnp.float32),
            jax.ShapeDtypeStruct((NPAD, 1), jnp.float32),
        ),
    )(degc, hw1)


def _tc_mid(acc, dinv, hw1, g1br, g2W):
    def body(a_ref, dinv_ref, hw1_ref, g1b_ref, g2w_ref, hws2_ref, hw2_ref):
        dinv = dinv_ref[...]
        h3 = _leaky(dinv * (a_ref[0, :, :EMB] + a_ref[1, :, :EMB])
                    + dinv * dinv * hw1_ref[...] + g1b_ref[...])
        hw2 = jnp.dot(h3, g2w_ref[...], preferred_element_type=jnp.float32)
        hw2_ref[...] = hw2
        hws2_ref[...] = dinv * hw2

    return pl.pallas_call(
        body,
        grid=(GRID,),
        in_specs=[_ACC, _ROWS_1, _ROWS_E, _full((1, EMB)),
                  _full((EMB, EMB))],
        out_specs=(_ROWS_E, _ROWS_E),
        out_shape=(
            jax.ShapeDtypeStruct((NPAD, EMB), jnp.float32),
            jax.ShapeDtypeStruct((NPAD, EMB), jnp.float32),
        ),
    )(acc, dinv, hw1, g1br, g2W)


def _tc_final(acc, dinv, hw2, g2br, cW, cbr):
    def body(a_ref, dinv_ref, hw2_ref, g2b_ref, cw_ref, cb_ref,
             hn_ref, ent_ref):
        dinv = dinv_ref[...]
        h4 = _leaky(dinv * (a_ref[0, :, :EMB] + a_ref[1, :, :EMB])
                    + dinv * dinv * hw2_ref[...] + g2b_ref[...])
        logits = jnp.dot(h4, cw_ref[...],
                         preferred_element_type=jnp.float32) + cb_ref[...]
        m = jnp.max(logits, axis=1, keepdims=True)
        z = logits - m
        lse = jnp.log(jnp.sum(jnp.exp(z), axis=1, keepdims=True))
        logp = z - lse
        p = jnp.exp(logp)
        ent_ref[...] = -jnp.sum(p * logp, axis=1, keepdims=True)
        nrm = jnp.maximum(
            jnp.sqrt(jnp.sum(h4 * h4, axis=1, keepdims=True)), 1e-8)
        hn_ref[...] = h4 / nrm

    return pl.pallas_call(
        body,
        grid=(GRID,),
        in_specs=[_ACC, _ROWS_1, _ROWS_E, _full((1, EMB)),
                  _full((EMB, CC)), _full((1, CC))],
        out_specs=(_ROWS_E, _ROWS_1),
        out_shape=(
            jax.ShapeDtypeStruct((NPAD, EMB), jnp.float32),
            jax.ShapeDtypeStruct((NPAD, 1), jnp.float32),
        ),
    )(acc, dinv, hw2, g2br, cW, cbr)


def _tc_score(agg, hn, ent, cnt):
    def body(a_ref, hn_ref, ent_ref, cnt_ref, out_ref):
        mask = (lax.broadcasted_iota(jnp.int32, (NPAD, 1), 0) < NN).astype(
            jnp.float32)
        ent = ent_ref[...]
        em = jnp.sum(ent * mask) / NN
        es = jnp.sqrt(jnp.sum((ent - em) ** 2 * mask) / (NN - 1))
        std_ent = (ent - em) / (es + 1e-8)
        cnt = cnt_ref[...]
        cm = jnp.sum(cnt * mask) / NN
        cs = jnp.sqrt(jnp.sum((cnt - cm) ** 2 * mask) / (NN - 1))
        gdd = GA * (cnt - cm) / (cs + 1e-8)
        ssum = jnp.sum(hn_ref[...] * (a_ref[0, :, :EMB] + a_ref[1, :, :EMB]),
                       axis=1, keepdims=True)
        avg = jnp.where(cnt > 0, ssum / jnp.maximum(cnt, 1.0), 1.0)
        out_ref[...] = AL * std_ent + BE * (1.0 - avg) + gdd

    return pl.pallas_call(
        body,
        out_shape=jax.ShapeDtypeStruct((NPAD, 1), jnp.float32),
    )(agg, hn, ent, cnt)


def kernel(x, edge_index, W1, b1, W2, b2, g1W, g1b, g2W, g2b, cW, cb):
    xp = jnp.pad(x, ((0, NPAD - NN), (0, 0)))
    pad_idx = NN + jnp.arange(EPAD - EE, dtype=jnp.int32) % (NPAD - NN)
    src = jnp.concatenate([edge_index[0], pad_idx]).reshape(NW, NCHUNK, CHUNK)
    dst = jnp.concatenate([edge_index[1], pad_idx]).reshape(NW, NCHUNK, CHUNK)
    zeros_n = jnp.zeros((NPAD,), jnp.float32)
    zeros_nd = jnp.zeros((NPAD, EMB), jnp.float32)
    ones_c = jnp.ones((CHUNK,), jnp.float32)
    b1r = b1.reshape(1, HH)
    b2r = b2.reshape(1, EMB)
    g1br = g1b.reshape(1, EMB)
    g2br = g2b.reshape(1, EMB)
    cbr = cb.reshape(1, CC)

    csrc_p, cdst_p = _sc_degree(src, dst, zeros_n, ones_c)
    hw1 = _tc_mlp(xp, W1, b1r, W2, b2r, g1W)
    degc = (cdst_p[0] + cdst_p[1]).reshape(NPAD, 1)
    cntc = degc + (csrc_p[0] + csrc_p[1]).reshape(NPAD, 1)
    hws1, dinv = _tc_prep(degc, hw1)
    acc1 = _sc_conv(src, dst, hws1, zeros_nd)
    hws2, hw2 = _tc_mid(acc1, dinv, hw1, g1br, g2W)
    acc2 = _sc_conv(src, dst, hws2, zeros_nd)
    hn, ent = _tc_final(acc2, dinv, hw2, g2br, cW, cbr)
    agg = _sc_sim(src, dst, hn, zeros_nd)
    score = _tc_score(agg, hn, ent, cntc)
    return score[:NN, 0]
